# initial kernel scaffold (unmeasured)
import jax
import jax.numpy as jnp
from jax import lax
from jax.experimental import pallas as pl
from jax.experimental.pallas import tpu as pltpu


def kernel(
    x,
):
    def body(*refs):
        pass

    out_shape = jax.ShapeDtypeStruct(..., jnp.float32)
    return pl.pallas_call(body, out_shape=out_shape)(...)


# baseline (device time: 110599 ns/iter reference)
import jax
import jax.numpy as jnp
from jax import lax
from jax.experimental import pallas as pl
from jax.experimental.pallas import tpu as pltpu


def kernel(x):
    m, n = x.shape
    half = m // 2

    def body(
        x_ref, out_ref, xtmp_ref, mine_ref, comm_ref,
        load_sem, store_sems, send_sems, recv_sems,
    ):
        my_x = lax.axis_index("x")
        my_y = lax.axis_index("y")

        barrier_sem = pltpu.get_barrier_semaphore()
        pl.semaphore_signal(
            barrier_sem, inc=1, device_id=(1 - my_x, my_y),
            device_id_type=pl.DeviceIdType.MESH,
        )
        pl.semaphore_signal(
            barrier_sem, inc=1, device_id=(my_x, 1 - my_y),
            device_id_type=pl.DeviceIdType.MESH,
        )
        pl.semaphore_wait(barrier_sem, 2)

        for h in range(2):
            load = pltpu.make_async_copy(
                x_ref.at[pl.ds(h * half, half)], xtmp_ref, load_sem
            )
            load.start()
            load.wait()
            mine_ref[pl.ds(h * half, half), :] = xtmp_ref[...].astype(
                jnp.bfloat16
            )

        store_mine = pltpu.make_async_copy(
            mine_ref, out_ref.at[pl.ds(my_x * m, m)], store_sems.at[0]
        )
        store_mine.start()

        rdma1 = pltpu.make_async_remote_copy(
            src_ref=mine_ref.at[pl.ds(my_y * half, half)],
            dst_ref=comm_ref.at[0],
            send_sem=send_sems.at[0],
            recv_sem=recv_sems.at[0],
            device_id=(1 - my_x, my_y),
            device_id_type=pl.DeviceIdType.MESH,
        )
        rdma1.start()
        rdma1.wait()

        rdma2 = pltpu.make_async_remote_copy(
            src_ref=comm_ref.at[0],
            dst_ref=comm_ref.at[1],
            send_sem=send_sems.at[1],
            recv_sem=recv_sems.at[1],
            device_id=(my_x, 1 - my_y),
            device_id_type=pl.DeviceIdType.MESH,
        )
        rdma2.start()
        rdma2.wait()

        other = 1 - my_x
        store_r0 = pltpu.make_async_copy(
            comm_ref.at[0],
            out_ref.at[pl.ds(other * m + my_y * half, half)],
            store_sems.at[1],
        )
        store_r0.start()
        store_r1 = pltpu.make_async_copy(
            comm_ref.at[1],
            out_ref.at[pl.ds(other * m + (1 - my_y) * half, half)],
            store_sems.at[2],
        )
        store_r1.start()
        store_mine.wait()
        store_r0.wait()
        store_r1.wait()

    return pl.pallas_call(
        body,
        out_shape=jax.ShapeDtypeStruct((2 * m, n), jnp.bfloat16),
        in_specs=[pl.BlockSpec(memory_space=pl.ANY)],
        out_specs=pl.BlockSpec(memory_space=pl.ANY),
        scratch_shapes=[
            pltpu.VMEM((half, n), jnp.float32),
            pltpu.VMEM((m, n), jnp.bfloat16),
            pltpu.VMEM((2, half, n), jnp.bfloat16),
            pltpu.SemaphoreType.DMA,
            pltpu.SemaphoreType.DMA((3,)),
            pltpu.SemaphoreType.DMA((2,)),
            pltpu.SemaphoreType.DMA((2,)),
        ],
        compiler_params=pltpu.CompilerParams(collective_id=0),
    )(x)


# device time: 63923 ns/iter; 1.7302x vs baseline; 1.7302x over previous
import jax
import jax.numpy as jnp
from jax import lax
from jax.experimental import pallas as pl
from jax.experimental.pallas import tpu as pltpu

NCHUNK = 8


def kernel(x):
    m, n = x.shape
    half = m // 2
    rows = half // NCHUNK

    def body(
        x_ref, out_ref, xtmp_ref, mine_ref, comm_ref,
        load_sems, store_sems, send_sems, recv_sems,
    ):
        my_x = lax.axis_index("x")
        my_y = lax.axis_index("y")
        other = 1 - my_x
        send_base = my_y * half
        keep_base = (1 - my_y) * half

        barrier_sem = pltpu.get_barrier_semaphore()
        pl.semaphore_signal(
            barrier_sem, inc=1, device_id=(1 - my_x, my_y),
            device_id_type=pl.DeviceIdType.MESH,
        )
        pl.semaphore_signal(
            barrier_sem, inc=1, device_id=(my_x, 1 - my_y),
            device_id_type=pl.DeviceIdType.MESH,
        )
        pl.semaphore_wait(barrier_sem, 2)

        loads = []
        for c in range(NCHUNK):
            ld = pltpu.make_async_copy(
                x_ref.at[pl.ds(send_base + c * rows, rows)],
                xtmp_ref.at[pl.ds(c * rows, rows)],
                load_sems.at[c],
            )
            ld.start()
            loads.append(ld)

        rdma1 = []
        for c in range(NCHUNK):
            loads[c].wait()
            mine_ref[pl.ds(send_base + c * rows, rows), :] = (
                xtmp_ref[pl.ds(c * rows, rows), :].astype(jnp.bfloat16)
            )
            r = pltpu.make_async_remote_copy(
                src_ref=mine_ref.at[pl.ds(send_base + c * rows, rows)],
                dst_ref=comm_ref.at[0, pl.ds(c * rows, rows)],
                send_sem=send_sems.at[0, c],
                recv_sem=recv_sems.at[0, c],
                device_id=(1 - my_x, my_y),
                device_id_type=pl.DeviceIdType.MESH,
            )
            r.start()
            rdma1.append(r)

        load_keep = pltpu.make_async_copy(
            x_ref.at[pl.ds(keep_base, half)],
            xtmp_ref,
            load_sems.at[NCHUNK],
        )
        load_keep.start()
        load_keep.wait()
        mine_ref[pl.ds(keep_base, half), :] = xtmp_ref[...].astype(
            jnp.bfloat16
        )
        store_mine = pltpu.make_async_copy(
            mine_ref, out_ref.at[pl.ds(my_x * m, m)], store_sems.at[2 * NCHUNK]
        )
        store_mine.start()

        rdma2 = []
        stores = []
        for c in range(NCHUNK):
            rdma1[c].wait_recv()
            r = pltpu.make_async_remote_copy(
                src_ref=comm_ref.at[0, pl.ds(c * rows, rows)],
                dst_ref=comm_ref.at[1, pl.ds(c * rows, rows)],
                send_sem=send_sems.at[1, c],
                recv_sem=recv_sems.at[1, c],
                device_id=(my_x, 1 - my_y),
                device_id_type=pl.DeviceIdType.MESH,
            )
            r.start()
            rdma2.append(r)
            st = pltpu.make_async_copy(
                comm_ref.at[0, pl.ds(c * rows, rows)],
                out_ref.at[pl.ds(other * m + my_y * half + c * rows, rows)],
                store_sems.at[c],
            )
            st.start()
            stores.append(st)

        for c in range(NCHUNK):
            rdma2[c].wait_recv()
            st = pltpu.make_async_copy(
                comm_ref.at[1, pl.ds(c * rows, rows)],
                out_ref.at[
                    pl.ds(other * m + (1 - my_y) * half + c * rows, rows)
                ],
                store_sems.at[NCHUNK + c],
            )
            st.start()
            stores.append(st)

        for c in range(NCHUNK):
            rdma1[c].wait_send()
            rdma2[c].wait_send()
        store_mine.wait()
        for st in stores:
            st.wait()

    return pl.pallas_call(
        body,
        out_shape=jax.ShapeDtypeStruct((2 * m, n), jnp.bfloat16),
        in_specs=[pl.BlockSpec(memory_space=pl.ANY)],
        out_specs=pl.BlockSpec(memory_space=pl.ANY),
        scratch_shapes=[
            pltpu.VMEM((half, n), jnp.float32),
            pltpu.VMEM((m, n), jnp.bfloat16),
            pltpu.VMEM((2, half, n), jnp.bfloat16),
            pltpu.SemaphoreType.DMA((NCHUNK + 1,)),
            pltpu.SemaphoreType.DMA((2 * NCHUNK + 1,)),
            pltpu.SemaphoreType.DMA((2, NCHUNK)),
            pltpu.SemaphoreType.DMA((2, NCHUNK)),
        ],
        compiler_params=pltpu.CompilerParams(collective_id=0),
    )(x)


# device time: 61884 ns/iter; 1.7872x vs baseline; 1.0329x over previous
import jax
import jax.numpy as jnp
from jax import lax
from jax.experimental import pallas as pl
from jax.experimental.pallas import tpu as pltpu

NCHUNK = 16


def kernel(x):
    m, n = x.shape
    half = m // 2
    rows = half // NCHUNK

    def body(
        x_ref, out_ref, xtmp_ref, mine_ref, comm_ref,
        load_sems, store_sems, send_sems, recv_sems,
    ):
        my_x = lax.axis_index("x")
        my_y = lax.axis_index("y")
        other = 1 - my_x
        send_base = my_y * half
        keep_base = (1 - my_y) * half

        barrier_sem = pltpu.get_barrier_semaphore()
        pl.semaphore_signal(
            barrier_sem, inc=1, device_id=(1 - my_x, my_y),
            device_id_type=pl.DeviceIdType.MESH,
        )
        pl.semaphore_signal(
            barrier_sem, inc=1, device_id=(my_x, 1 - my_y),
            device_id_type=pl.DeviceIdType.MESH,
        )
        pl.semaphore_wait(barrier_sem, 2)

        loads = []
        for c in range(NCHUNK):
            ld = pltpu.make_async_copy(
                x_ref.at[pl.ds(send_base + c * rows, rows)],
                xtmp_ref.at[pl.ds(c * rows, rows)],
                load_sems.at[c],
            )
            ld.start()
            loads.append(ld)

        rdma1 = []
        for c in range(NCHUNK):
            loads[c].wait()
            mine_ref[pl.ds(send_base + c * rows, rows), :] = (
                xtmp_ref[pl.ds(c * rows, rows), :].astype(jnp.bfloat16)
            )
            r = pltpu.make_async_remote_copy(
                src_ref=mine_ref.at[pl.ds(send_base + c * rows, rows)],
                dst_ref=comm_ref.at[0, pl.ds(c * rows, rows)],
                send_sem=send_sems.at[0, c],
                recv_sem=recv_sems.at[0, c],
                device_id=(1 - my_x, my_y),
                device_id_type=pl.DeviceIdType.MESH,
            )
            r.start()
            rdma1.append(r)

        keep_loads = []
        for c in range(NCHUNK):
            ld = pltpu.make_async_copy(
                x_ref.at[pl.ds(keep_base + c * rows, rows)],
                xtmp_ref.at[pl.ds(c * rows, rows)],
                load_sems.at[NCHUNK + c],
            )
            ld.start()
            keep_loads.append(ld)

        rdma2 = []
        stores = []
        for c in range(NCHUNK):
            keep_loads[c].wait()
            mine_ref[pl.ds(keep_base + c * rows, rows), :] = (
                xtmp_ref[pl.ds(c * rows, rows), :].astype(jnp.bfloat16)
            )
            rdma1[c].wait_recv()
            r = pltpu.make_async_remote_copy(
                src_ref=comm_ref.at[0, pl.ds(c * rows, rows)],
                dst_ref=comm_ref.at[1, pl.ds(c * rows, rows)],
                send_sem=send_sems.at[1, c],
                recv_sem=recv_sems.at[1, c],
                device_id=(my_x, 1 - my_y),
                device_id_type=pl.DeviceIdType.MESH,
            )
            r.start()
            rdma2.append(r)
            st = pltpu.make_async_copy(
                comm_ref.at[0, pl.ds(c * rows, rows)],
                out_ref.at[pl.ds(other * m + my_y * half + c * rows, rows)],
                store_sems.at[c],
            )
            st.start()
            stores.append(st)

        store_mine = pltpu.make_async_copy(
            mine_ref, out_ref.at[pl.ds(my_x * m, m)], store_sems.at[2 * NCHUNK]
        )
        store_mine.start()

        for c in range(NCHUNK):
            rdma2[c].wait_recv()
            st = pltpu.make_async_copy(
                comm_ref.at[1, pl.ds(c * rows, rows)],
                out_ref.at[
                    pl.ds(other * m + (1 - my_y) * half + c * rows, rows)
                ],
                store_sems.at[NCHUNK + c],
            )
            st.start()
            stores.append(st)

        for c in range(NCHUNK):
            rdma1[c].wait_send()
            rdma2[c].wait_send()
        store_mine.wait()
        for st in stores:
            st.wait()

    return pl.pallas_call(
        body,
        out_shape=jax.ShapeDtypeStruct((2 * m, n), jnp.bfloat16),
        in_specs=[pl.BlockSpec(memory_space=pl.ANY)],
        out_specs=pl.BlockSpec(memory_space=pl.ANY),
        scratch_shapes=[
            pltpu.VMEM((half, n), jnp.float32),
            pltpu.VMEM((m, n), jnp.bfloat16),
            pltpu.VMEM((2, half, n), jnp.bfloat16),
            pltpu.SemaphoreType.DMA((2 * NCHUNK,)),
            pltpu.SemaphoreType.DMA((2 * NCHUNK + 1,)),
            pltpu.SemaphoreType.DMA((2, NCHUNK)),
            pltpu.SemaphoreType.DMA((2, NCHUNK)),
        ],
        compiler_params=pltpu.CompilerParams(collective_id=0),
    )(x)


# device time: 61572 ns/iter; 1.7963x vs baseline; 1.0051x over previous
import jax
import jax.numpy as jnp
from jax import lax
from jax.experimental import pallas as pl
from jax.experimental.pallas import tpu as pltpu

NCHUNK = 16


def kernel(x):
    m, n = x.shape
    half = m // 2
    rows = half // NCHUNK

    def body(
        x_ref, out_ref, xtmp_ref, mine_ref, comm_ref,
        load_sems, store_sems, send_sems, recv_sems,
    ):
        my_x = lax.axis_index("x")
        my_y = lax.axis_index("y")
        other = 1 - my_x
        send_base = my_y * half
        keep_base = (1 - my_y) * half

        loads = []
        for c in range(NCHUNK):
            ld = pltpu.make_async_copy(
                x_ref.at[pl.ds(send_base + c * rows, rows)],
                xtmp_ref.at[pl.ds(c * rows, rows)],
                load_sems.at[c],
            )
            ld.start()
            loads.append(ld)

        barrier_sem = pltpu.get_barrier_semaphore()
        pl.semaphore_signal(
            barrier_sem, inc=1, device_id=(1 - my_x, my_y),
            device_id_type=pl.DeviceIdType.MESH,
        )
        pl.semaphore_signal(
            barrier_sem, inc=1, device_id=(my_x, 1 - my_y),
            device_id_type=pl.DeviceIdType.MESH,
        )
        pl.semaphore_wait(barrier_sem, 2)

        rdma1 = []
        keep_loads = []
        for c in range(NCHUNK):
            loads[c].wait()
            mine_ref[pl.ds(send_base + c * rows, rows), :] = (
                xtmp_ref[pl.ds(c * rows, rows), :].astype(jnp.bfloat16)
            )
            r = pltpu.make_async_remote_copy(
                src_ref=mine_ref.at[pl.ds(send_base + c * rows, rows)],
                dst_ref=comm_ref.at[0, pl.ds(c * rows, rows)],
                send_sem=send_sems.at[0, c],
                recv_sem=recv_sems.at[0, c],
                device_id=(1 - my_x, my_y),
                device_id_type=pl.DeviceIdType.MESH,
            )
            r.start()
            rdma1.append(r)
            kl = pltpu.make_async_copy(
                x_ref.at[pl.ds(keep_base + c * rows, rows)],
                xtmp_ref.at[pl.ds(c * rows, rows)],
                load_sems.at[NCHUNK + c],
            )
            kl.start()
            keep_loads.append(kl)

        rdma2 = []
        stores = []
        for c in range(NCHUNK):
            rdma1[c].wait_recv()
            r = pltpu.make_async_remote_copy(
                src_ref=comm_ref.at[0, pl.ds(c * rows, rows)],
                dst_ref=comm_ref.at[1, pl.ds(c * rows, rows)],
                send_sem=send_sems.at[1, c],
                recv_sem=recv_sems.at[1, c],
                device_id=(my_x, 1 - my_y),
                device_id_type=pl.DeviceIdType.MESH,
            )
            r.start()
            rdma2.append(r)
            st = pltpu.make_async_copy(
                comm_ref.at[0, pl.ds(c * rows, rows)],
                out_ref.at[pl.ds(other * m + my_y * half + c * rows, rows)],
                store_sems.at[c],
            )
            st.start()
            stores.append(st)
            keep_loads[c].wait()
            mine_ref[pl.ds(keep_base + c * rows, rows), :] = (
                xtmp_ref[pl.ds(c * rows, rows), :].astype(jnp.bfloat16)
            )

        store_mine = pltpu.make_async_copy(
            mine_ref, out_ref.at[pl.ds(my_x * m, m)], store_sems.at[2 * NCHUNK]
        )
        store_mine.start()

        for c in range(NCHUNK):
            rdma2[c].wait_recv()
            st = pltpu.make_async_copy(
                comm_ref.at[1, pl.ds(c * rows, rows)],
                out_ref.at[
                    pl.ds(other * m + (1 - my_y) * half + c * rows, rows)
                ],
                store_sems.at[NCHUNK + c],
            )
            st.start()
            stores.append(st)

        for c in range(NCHUNK):
            rdma1[c].wait_send()
            rdma2[c].wait_send()
        store_mine.wait()
        for st in stores:
            st.wait()

    return pl.pallas_call(
        body,
        out_shape=jax.ShapeDtypeStruct((2 * m, n), jnp.bfloat16),
        in_specs=[pl.BlockSpec(memory_space=pl.ANY)],
        out_specs=pl.BlockSpec(memory_space=pl.ANY),
        scratch_shapes=[
            pltpu.VMEM((half, n), jnp.float32),
            pltpu.VMEM((m, n), jnp.bfloat16),
            pltpu.VMEM((2, half, n), jnp.bfloat16),
            pltpu.SemaphoreType.DMA((2 * NCHUNK,)),
            pltpu.SemaphoreType.DMA((2 * NCHUNK + 1,)),
            pltpu.SemaphoreType.DMA((2, NCHUNK)),
            pltpu.SemaphoreType.DMA((2, NCHUNK)),
        ],
        compiler_params=pltpu.CompilerParams(collective_id=0),
    )(x)
